# baseline (speedup 1.0000x reference)
import jax
import jax.numpy as jnp
from jax.experimental import pallas as pl
from jax.experimental.pallas import tpu as pltpu

B = 16384
COND_DIM = 256
ACT_DIM = 32
DIN = COND_DIM + ACT_DIM
H0, H1 = 512, 256
EMB = 64
K = 1024
COMMIT = 0.25

BLK = 4096
NB = B // BLK
RG = BLK // 8


def _elu(x):
    return jnp.where(x > 0, x, jnp.exp(x) - jnp.ones((), x.dtype))


def _dot(a, b, dims):
    return jax.lax.dot_general(a, b, (dims, ((), ())),
                               preferred_element_type=jnp.float32)


def _body(xt_ref, We0_ref, be0_ref, We1_ref, be1_ref, We2t_ref, be2_ref,
          Wd0_ref, bd0_ref, Wd1_ref, bd1_ref, Wd2t_ref, bd2_ref, cbt_ref,
          recont_ref, idx_ref, stats_ref,
          counts_ref, qacc_ref, racc_ref, cbnt_ref, cbtb_ref):
    i = pl.program_id(0)

    @pl.when(i == 0)
    def _init():
        counts_ref[...] = jnp.zeros_like(counts_ref)
        qacc_ref[...] = jnp.zeros_like(qacc_ref)
        racc_ref[...] = jnp.zeros_like(racc_ref)
        cbt0 = cbt_ref[...]
        cbnt_ref[...] = cbt0 / (jnp.sqrt(jnp.sum(cbt0 * cbt0, axis=0,
                                                 keepdims=True)) + 1e-12)
        cbtb_ref[...] = cbt0.astype(jnp.bfloat16)

    xt = xt_ref[...]
    act_t = xt[COND_DIM:, :]

    h = _elu(_dot(xt, We0_ref[...], ((0,), (0,))) + be0_ref[...])
    h = _elu(jnp.dot(h, We1_ref[...], preferred_element_type=jnp.float32)
             + be1_ref[...])
    z = _dot(h, We2t_ref[...], ((1,), (1,))) + be2_ref[...]

    zn = z / (jnp.sqrt(jnp.sum(z * z, axis=-1, keepdims=True)) + 1e-12)
    dist = _dot(zn, cbnt_ref[...], ((1,), (0,)))

    idx = jnp.argmax(dist, axis=-1).astype(jnp.int32)
    idx_ref[...] = idx

    onehot = (jax.lax.broadcasted_iota(jnp.int32, (BLK, K), 1)
              == idx[:, None]).astype(jnp.bfloat16)
    oh4 = onehot.reshape(2, RG // 16, 8, 8, K)
    c8 = (jnp.sum(oh4[0], axis=(0, 1)).astype(jnp.float32)
          + jnp.sum(oh4[1], axis=(0, 1)).astype(jnp.float32))
    counts_ref[...] += c8

    quantized = jax.lax.dot_general(
        onehot, cbtb_ref[...], ((((1,), (1,))), ((), ())),
        preferred_element_type=jnp.float32)
    qd = quantized - z
    qacc_ref[...] += jnp.sum((qd * qd).reshape(RG, 8, EMB), axis=0)

    d = _elu(_dot(xt[:COND_DIM, :], Wd0_ref[:COND_DIM, :], ((0,), (0,)))
             + _dot(quantized, Wd0_ref[COND_DIM:, :], ((1,), (0,)))
             + bd0_ref[...])
    d = _elu(_dot(d, Wd1_ref[...], ((1,), (0,))) + bd1_ref[...])
    recon_t = (_dot(Wd2t_ref[...], d, ((1,), (1,)))
               + bd2_ref[...][:, None])
    recont_ref[...] = recon_t

    rd = recon_t - act_t
    racc_ref[...] += jnp.sum((rd * rd).reshape(4, 8, BLK // 128, 128),
                             axis=(0, 2))

    @pl.when(i == NB - 1)
    def _finalize():
        q_loss = jnp.sum(qacc_ref[...]) / (B * EMB)
        rec_loss = jnp.sum(racc_ref[...]) / (B * ACT_DIM)
        p = jnp.sum(counts_ref[...], axis=0, keepdims=True) / B
        perp = jnp.exp(-jnp.sum(p * jnp.log(p + 1e-10)))
        lane = jax.lax.broadcasted_iota(jnp.int32, (1, 128), 1)
        out = jnp.where(lane == 0, q_loss,
              jnp.where(lane == 1, COMMIT * q_loss,
              jnp.where(lane == 2, rec_loss, perp)))
        stats_ref[...] = out


def kernel(input, We0, be0, We1, be1, We2, be2,
           Wd0, bd0, Wd1, bd1, Wd2, bd2, codebook):
    full = lambda shape: pl.BlockSpec(shape, lambda i: (0,) * len(shape))
    recon_t, idx, stats = pl.pallas_call(
        _body,
        grid=(NB,),
        in_specs=[
            pl.BlockSpec((DIN, BLK), lambda i: (0, i)),
            full((DIN, H0)), full((H0,)),
            full((H0, H1)), full((H1,)),
            full((EMB, H1)), full((EMB,)),
            full((COND_DIM + EMB, H1)), full((H1,)),
            full((H1, H0)), full((H0,)),
            full((ACT_DIM, H0)), full((ACT_DIM,)),
            full((EMB, K)),
        ],
        out_specs=[
            pl.BlockSpec((ACT_DIM, BLK), lambda i: (0, i)),
            pl.BlockSpec((BLK,), lambda i: (i,)),
            pl.BlockSpec((1, 128), lambda i: (0, 0)),
        ],
        out_shape=[
            jax.ShapeDtypeStruct((ACT_DIM, B), jnp.float32),
            jax.ShapeDtypeStruct((B,), jnp.int32),
            jax.ShapeDtypeStruct((1, 128), jnp.float32),
        ],
        scratch_shapes=[
            pltpu.VMEM((8, K), jnp.float32),
            pltpu.VMEM((8, EMB), jnp.float32),
            pltpu.VMEM((8, 128), jnp.float32),
            pltpu.VMEM((EMB, K), jnp.float32),
            pltpu.VMEM((EMB, K), jnp.bfloat16),
        ],
    )(input.T, We0, be0, We1, be1, We2.T, be2,
      Wd0, bd0, Wd1, bd1, Wd2.T, bd2, codebook.T)
    return (recon_t.T, idx, stats[0, 0], stats[0, 1], stats[0, 2],
            stats[0, 3])


# speedup vs baseline: 1.0163x; 1.0163x over previous
import jax
import jax.numpy as jnp
from jax.experimental import pallas as pl
from jax.experimental.pallas import tpu as pltpu

B = 16384
COND_DIM = 256
ACT_DIM = 32
DIN = COND_DIM + ACT_DIM
H0, H1 = 512, 256
EMB = 64
K = 1024
COMMIT = 0.25

BLK = 4096
NB = B // BLK
RG = BLK // 8


def _elu(x):
    return jnp.where(x > 0, x, jnp.exp(x) - jnp.ones((), x.dtype))


def _dot(a, b, dims):
    return jax.lax.dot_general(a, b, (dims, ((), ())),
                               preferred_element_type=jnp.float32)


def _body(xt_ref, We0_ref, be0_ref, We1_ref, be1_ref, We2t_ref, be2_ref,
          Wd0_ref, bd0_ref, Wd1_ref, bd1_ref, Wd2t_ref, bd2_ref, cbt_ref,
          recont_ref, idx_ref, stats_ref,
          counts_ref, qacc_ref, racc_ref, cbnt_ref, cbtb_ref):
    i = pl.program_id(0)

    @pl.when(i == 0)
    def _init():
        counts_ref[...] = jnp.zeros_like(counts_ref)
        qacc_ref[...] = jnp.zeros_like(qacc_ref)
        racc_ref[...] = jnp.zeros_like(racc_ref)
        cbt0 = cbt_ref[...]
        cbnt_ref[...] = cbt0 / (jnp.sqrt(jnp.sum(cbt0 * cbt0, axis=0,
                                                 keepdims=True)) + 1e-12)
        cbtb_ref[...] = cbt0.astype(jnp.bfloat16)

    xt = xt_ref[...]
    act_t = xt[COND_DIM:, :]

    h = _elu(_dot(xt, We0_ref[...], ((0,), (0,))) + be0_ref[...])
    h = _elu(_dot(h, We1_ref[...], ((1,), (0,))) + be1_ref[...])
    z = _dot(h, We2t_ref[...], ((1,), (1,))) + be2_ref[...]

    zn = z / (jnp.sqrt(jnp.sum(z * z, axis=-1, keepdims=True)) + 1e-12)
    dist = _dot(zn, cbnt_ref[...], ((1,), (0,)))

    idx = jnp.argmax(dist, axis=-1).astype(jnp.int32)
    idx_ref[...] = idx

    onehot = (jax.lax.broadcasted_iota(jnp.int32, (BLK, K), 1)
              == idx[:, None]).astype(jnp.bfloat16)
    oh4 = onehot.reshape(2, RG // 16, 8, 8, K)
    c8 = (jnp.sum(oh4[0], axis=(0, 1)).astype(jnp.float32)
          + jnp.sum(oh4[1], axis=(0, 1)).astype(jnp.float32))
    counts_ref[...] += c8

    quantized = jax.lax.dot_general(
        onehot, cbtb_ref[...], ((((1,), (1,))), ((), ())),
        preferred_element_type=jnp.float32)
    qd = quantized - z
    qacc_ref[...] += jnp.sum((qd * qd).reshape(RG, 8, EMB), axis=0)

    def _bdot(a, b, dims):
        return jax.lax.dot_general(
            a.astype(jnp.bfloat16), b.astype(jnp.bfloat16),
            (dims, ((), ())), preferred_element_type=jnp.float32)

    d = _elu(_bdot(xt[:COND_DIM, :], Wd0_ref[:COND_DIM, :], ((0,), (0,)))
             + _bdot(quantized, Wd0_ref[COND_DIM:, :], ((1,), (0,)))
             + bd0_ref[...])
    d = _elu(_bdot(d, Wd1_ref[...], ((1,), (0,))) + bd1_ref[...])
    recon_t = (_bdot(Wd2t_ref[...], d, ((1,), (1,)))
               + bd2_ref[...][:, None])
    recont_ref[...] = recon_t

    rd = recon_t - act_t
    racc_ref[...] += jnp.sum((rd * rd).reshape(4, 8, BLK // 128, 128),
                             axis=(0, 2))

    @pl.when(i == NB - 1)
    def _finalize():
        q_loss = jnp.sum(qacc_ref[...]) / (B * EMB)
        rec_loss = jnp.sum(racc_ref[...]) / (B * ACT_DIM)
        p = jnp.sum(counts_ref[...], axis=0, keepdims=True) / B
        perp = jnp.exp(-jnp.sum(p * jnp.log(p + 1e-10)))
        lane = jax.lax.broadcasted_iota(jnp.int32, (1, 128), 1)
        out = jnp.where(lane == 0, q_loss,
              jnp.where(lane == 1, COMMIT * q_loss,
              jnp.where(lane == 2, rec_loss, perp)))
        stats_ref[...] = out


def kernel(input, We0, be0, We1, be1, We2, be2,
           Wd0, bd0, Wd1, bd1, Wd2, bd2, codebook):
    full = lambda shape: pl.BlockSpec(shape, lambda i: (0,) * len(shape))
    recon_t, idx, stats = pl.pallas_call(
        _body,
        grid=(NB,),
        in_specs=[
            pl.BlockSpec((DIN, BLK), lambda i: (0, i)),
            full((DIN, H0)), full((H0,)),
            full((H0, H1)), full((H1,)),
            full((EMB, H1)), full((EMB,)),
            full((COND_DIM + EMB, H1)), full((H1,)),
            full((H1, H0)), full((H0,)),
            full((ACT_DIM, H0)), full((ACT_DIM,)),
            full((EMB, K)),
        ],
        out_specs=[
            pl.BlockSpec((ACT_DIM, BLK), lambda i: (0, i)),
            pl.BlockSpec((BLK,), lambda i: (i,)),
            pl.BlockSpec((1, 128), lambda i: (0, 0)),
        ],
        out_shape=[
            jax.ShapeDtypeStruct((ACT_DIM, B), jnp.float32),
            jax.ShapeDtypeStruct((B,), jnp.int32),
            jax.ShapeDtypeStruct((1, 128), jnp.float32),
        ],
        scratch_shapes=[
            pltpu.VMEM((8, K), jnp.float32),
            pltpu.VMEM((8, EMB), jnp.float32),
            pltpu.VMEM((8, 128), jnp.float32),
            pltpu.VMEM((EMB, K), jnp.float32),
            pltpu.VMEM((EMB, K), jnp.bfloat16),
        ],
    )(input.T, We0, be0, We1, be1, We2.T, be2,
      Wd0, bd0, Wd1, bd1, Wd2.T, bd2, codebook.T)
    return (recon_t.T, idx, stats[0, 0], stats[0, 1], stats[0, 2],
            stats[0, 3])
